# baseline (speedup 1.0000x reference)
import functools

import jax
import jax.numpy as jnp
from jax import lax
from jax.experimental import pallas as pl
from jax.experimental.pallas import tpu as pltpu
from jax.experimental.pallas import tpu_sc as plsc

V, D = 1000000, 32
DH = D // 2
N, K = 16384, 20
B = N * K
NC, NS = 2, 16
NW = NC * NS
LANES = 16

_mesh = plsc.VectorSubcoreMesh(core_axis_name="c", subcore_axis_name="s")

CHUNK = 256
NBLK = V // CHUNK
TAIL = V - NBLK * CHUNK
NF = NBLK // NW
NREM = NBLK - NF * NW
QPB = CHUNK * DH // 128


@functools.partial(
    pl.kernel,
    mesh=_mesh,
    out_type=jax.ShapeDtypeStruct((V * DH // 128, 128), jnp.float32),
    scratch_types=[
        pltpu.VMEM((2, CHUNK, DH), jnp.float32),
        pltpu.VMEM((2, QPB, 128), jnp.float32),
        pltpu.SemaphoreType.DMA,
        pltpu.SemaphoreType.DMA,
        pltpu.SemaphoreType.DMA,
        pltpu.SemaphoreType.DMA,
    ],
    compiler_params=pltpu.CompilerParams(use_tc_tiling_on_sc=True),
)
def _pack_kernel(tab_hbm, rm_hbm, vbuf, pbuf, sem_i0, sem_i1, sem_o0, sem_o1):
    wid = lax.axis_index("s") * NC + lax.axis_index("c")
    sem_i = (sem_i0, sem_i1)
    sem_o = (sem_o0, sem_o1)

    def pack_block(p, nrows):
        for r in range(nrows):
            vec = vbuf[p, r, pl.ds(0, LANES)]
            pbuf[p, r // 8, pl.ds((r % 8) * DH, LANES)] = vec

    def start_in(b, p):
        return pltpu.async_copy(
            tab_hbm.at[pl.ds(b * CHUNK, CHUNK)], vbuf.at[p], sem_i[p])

    def wait_in(p):
        pltpu.make_async_copy(
            tab_hbm.at[pl.ds(0, CHUNK)], vbuf.at[p], sem_i[p]).wait()

    def start_out(b, p):
        return pltpu.async_copy(
            pbuf.at[p], rm_hbm.at[pl.ds(b * QPB, QPB)], sem_o[p])

    def wait_out(p):
        pltpu.make_async_copy(
            pbuf.at[p], rm_hbm.at[pl.ds(0, QPB)], sem_o[p]).wait()

    start_in(wid, 0)
    start_in(wid + NW, 1)

    def body(i2, _):
        i = i2 * 2
        for p in (0, 1):
            wait_in(p)

            @pl.when(i + p >= 2)
            def _():
                wait_out(p)

            pack_block(p, CHUNK)
            start_out(wid + (i + p) * NW, p)

            @pl.when(i + p + 2 < NF)
            def _():
                start_in(wid + (i + p + 2) * NW, p)

        return _

    lax.fori_loop(0, NF // 2, body, None)
    wait_out(0)
    wait_out(1)

    @pl.when(wid < NREM)
    def _():
        b = NF * NW + wid
        pltpu.sync_copy(tab_hbm.at[pl.ds(b * CHUNK, CHUNK)], vbuf.at[0])
        pack_block(0, CHUNK)
        pltpu.sync_copy(pbuf.at[0], rm_hbm.at[pl.ds(b * QPB, QPB)])

    @pl.when(wid == NREM)
    def _():
        r0 = NBLK * CHUNK
        pltpu.sync_copy(tab_hbm.at[pl.ds(r0, TAIL)], vbuf.at[0, pl.ds(0, TAIL)])
        pack_block(0, TAIL)
        pltpu.sync_copy(pbuf.at[0, pl.ds(0, TAIL * DH // 128)],
                        rm_hbm.at[pl.ds(r0 * DH // 128, TAIL * DH // 128)])


B_PER_W = B // NW
CH = 1024
NCHUNK = B_PER_W // CH


@functools.partial(
    pl.kernel,
    mesh=_mesh,
    out_type=jax.ShapeDtypeStruct((B, D), jnp.float32),
    scratch_types=[
        pltpu.VMEM((NCHUNK, CH), jnp.int32),
        pltpu.VMEM((2, CH, DH), jnp.float32),
        pltpu.VMEM((2, CH, DH), jnp.float32),
        pltpu.SemaphoreType.DMA,
        pltpu.SemaphoreType.DMA,
        pltpu.SemaphoreType.DMA,
        pltpu.SemaphoreType.DMA,
    ],
    compiler_params=pltpu.CompilerParams(use_tc_tiling_on_sc=False),
)
def _gather_kernel(idx_hbm, tlo_hbm, thi_hbm, out_hbm, idx_v, rows_lo, rows_hi,
                   sem_g0, sem_g1, sem_w0, sem_w1):
    wid = lax.axis_index("s") * NC + lax.axis_index("c")
    base = wid * B_PER_W
    sem_g = (sem_g0, sem_g1)
    sem_w = (sem_w0, sem_w1)

    pltpu.sync_copy(idx_hbm.at[wid], idx_v)

    def start_gather(c, b):
        h1 = pltpu.async_copy(tlo_hbm.at[idx_v.at[c]], rows_lo.at[b], sem_g[b])
        h2 = pltpu.async_copy(thi_hbm.at[idx_v.at[c]], rows_hi.at[b], sem_g[b])
        return (h1, h2)

    def start_write(c, b):
        rows = pl.ds(base + c * CH, CH)
        h1 = pltpu.async_copy(
            rows_lo.at[b], out_hbm.at[rows, pl.ds(0, DH)], sem_w[b])
        h2 = pltpu.async_copy(
            rows_hi.at[b], out_hbm.at[rows, pl.ds(DH, DH)], sem_w[b])
        return (h1, h2)

    h_g = [None, None]
    h_w = [None, None]
    h_g[0] = start_gather(0, 0)
    for c in range(NCHUNK):
        b = c % 2
        nb = (c + 1) % 2
        if c + 1 < NCHUNK:
            if h_w[nb] is not None:
                for h in h_w[nb]:
                    h.wait()
                h_w[nb] = None
            h_g[nb] = start_gather(c + 1, nb)
        for h in h_g[b]:
            h.wait()
        h_w[b] = start_write(c, b)
    for b in range(2):
        if h_w[b] is not None:
            for h in h_w[b]:
                h.wait()


def kernel(index, table):
    tlo = lax.slice(table, (0, 0), (V, DH))
    thi_rm = _pack_kernel(lax.slice(table, (0, DH), (V, D)))
    thi = thi_rm.reshape(V, DH)
    idx = index.reshape(-1).astype(jnp.int32).reshape(NW, NCHUNK, CH)
    out = _gather_kernel(idx, tlo, thi)
    return out.reshape(index.shape[0], index.shape[1], D)


# speedup vs baseline: 1.7412x; 1.7412x over previous
import functools

import jax
import jax.numpy as jnp
from jax import lax
from jax.experimental import pallas as pl
from jax.experimental.pallas import tpu as pltpu
from jax.experimental.pallas import tpu_sc as plsc

B = 16384 * 20
D = 32
NC, NS = 2, 16
NW = NC * NS
B_PER_W = B // NW
CH = 1024
NCHUNK = B_PER_W // CH

_mesh = plsc.VectorSubcoreMesh(core_axis_name="c", subcore_axis_name="s")


@functools.partial(
    pl.kernel,
    mesh=_mesh,
    out_type=jax.ShapeDtypeStruct((B, D), jnp.float32),
    scratch_types=[
        pltpu.VMEM((NCHUNK, CH), jnp.int32),
        pltpu.VMEM((2, CH, D), jnp.float32),
        pltpu.SemaphoreType.DMA,
        pltpu.SemaphoreType.DMA,
        pltpu.SemaphoreType.DMA,
        pltpu.SemaphoreType.DMA,
    ],
    compiler_params=pltpu.CompilerParams(use_tc_tiling_on_sc=False),
)
def _gather_kernel(idx_hbm, table_hbm, out_hbm, idx_v, rows_v,
                   sem_g0, sem_g1, sem_w0, sem_w1):
    wid = lax.axis_index("s") * NC + lax.axis_index("c")
    base = wid * B_PER_W
    sem_g = (sem_g0, sem_g1)
    sem_w = (sem_w0, sem_w1)

    pltpu.sync_copy(idx_hbm.at[wid], idx_v)

    h_g = [None, None]
    h_w = [None, None]
    h_g[0] = pltpu.async_copy(table_hbm.at[idx_v.at[0]], rows_v.at[0], sem_g[0])
    for c in range(NCHUNK):
        b = c % 2
        nb = (c + 1) % 2
        if c + 1 < NCHUNK:
            if h_w[nb] is not None:
                h_w[nb].wait()
                h_w[nb] = None
            h_g[nb] = pltpu.async_copy(
                table_hbm.at[idx_v.at[c + 1]], rows_v.at[nb], sem_g[nb])
        h_g[b].wait()
        h_w[b] = pltpu.async_copy(
            rows_v.at[b], out_hbm.at[pl.ds(base + c * CH, CH)], sem_w[b])
    for b in range(2):
        if h_w[b] is not None:
            h_w[b].wait()


def kernel(index, table):
    idx = index.reshape(-1).astype(jnp.int32).reshape(NW, NCHUNK, CH)
    out = _gather_kernel(idx, table)
    return out.reshape(index.shape[0], index.shape[1], D)
